# baseline (speedup 1.0000x reference)
import jax
import jax.numpy as jnp
from jax.experimental import pallas as pl
from jax.experimental.pallas import tpu as pltpu

JOINTS = 22
EMB = 2048
NQ = 8
RQ = EMB // NQ


def _edge_enhance_kernel(src_ref, b_ref, w_hbm, out_ref, w_vmem, sems):
    copies = []
    for q in range(NQ):
        cp = pltpu.make_async_copy(
            w_hbm.at[pl.ds(q * RQ, RQ), :], w_vmem.at[q], sems.at[q])
        cp.start()
        copies.append(cp)
    for q in range(NQ):
        copies[q].wait()
        y = jax.lax.dot_general(
            src_ref[...], w_vmem[q],
            dimension_numbers=(((1,), (1,)), ((), ())),
            preferred_element_type=jnp.float32,
        )
        yb = y + b_ref[:, q * RQ:(q + 1) * RQ]
        out_ref[:, :, q * RQ:(q + 1) * RQ] = yb[None, :, :] - y[:, None, :]


def kernel(src, W, b):
    src0 = src[0]
    b2d = b.reshape(1, EMB)
    out = pl.pallas_call(
        _edge_enhance_kernel,
        in_specs=[
            pl.BlockSpec((JOINTS, EMB), lambda: (0, 0)),
            pl.BlockSpec((1, EMB), lambda: (0, 0)),
            pl.BlockSpec(memory_space=pltpu.MemorySpace.HBM),
        ],
        out_specs=pl.BlockSpec((JOINTS, JOINTS, EMB), lambda: (0, 0, 0)),
        out_shape=jax.ShapeDtypeStruct((JOINTS, JOINTS, EMB), jnp.float32),
        scratch_shapes=[
            pltpu.VMEM((NQ, RQ, EMB), jnp.float32),
            pltpu.SemaphoreType.DMA((NQ,)),
        ],
    )(src0, b2d, W)
    return out


# speedup vs baseline: 1.1098x; 1.1098x over previous
import jax
import jax.numpy as jnp
from jax.experimental import pallas as pl
from jax.experimental.pallas import tpu as pltpu

JOINTS = 22
EMB = 2048
NQ = 4
RQ = EMB // NQ


def _edge_enhance_kernel(src_ref, b_ref, w_hbm, out_hbm, w_vmem, out_vmem,
                         in_sems, out_sems):
    copies = []
    for q in range(NQ):
        cp = pltpu.make_async_copy(
            w_hbm.at[pl.ds(q * RQ, RQ), :], w_vmem.at[q], in_sems.at[q])
        cp.start()
        copies.append(cp)
    stores = []
    for q in range(NQ):
        copies[q].wait()
        y = jax.lax.dot_general(
            src_ref[...], w_vmem[q],
            dimension_numbers=(((1,), (1,)), ((), ())),
            preferred_element_type=jnp.float32,
        )
        yb = y + b_ref[:, q * RQ:(q + 1) * RQ]
        out_vmem[q] = yb[None, :, :] - y[:, None, :]
        st = pltpu.make_async_copy(
            out_vmem.at[q],
            out_hbm.at[:, :, pl.ds(q * RQ, RQ)],
            out_sems.at[q])
        st.start()
        stores.append(st)
    for st in stores:
        st.wait()


def kernel(src, W, b):
    src0 = src[0]
    b2d = b.reshape(1, EMB)
    out = pl.pallas_call(
        _edge_enhance_kernel,
        in_specs=[
            pl.BlockSpec((JOINTS, EMB), lambda: (0, 0)),
            pl.BlockSpec((1, EMB), lambda: (0, 0)),
            pl.BlockSpec(memory_space=pltpu.MemorySpace.HBM),
        ],
        out_specs=pl.BlockSpec(memory_space=pltpu.MemorySpace.HBM),
        out_shape=jax.ShapeDtypeStruct((JOINTS, JOINTS, EMB), jnp.float32),
        scratch_shapes=[
            pltpu.VMEM((NQ, RQ, EMB), jnp.float32),
            pltpu.VMEM((NQ, JOINTS, JOINTS, RQ), jnp.float32),
            pltpu.SemaphoreType.DMA((NQ,)),
            pltpu.SemaphoreType.DMA((NQ,)),
        ],
    )(src0, b2d, W)
    return out
